# baseline (speedup 1.0000x reference)
import functools

import jax
import jax.numpy as jnp
from jax import lax
from jax.experimental import pallas as pl
from jax.experimental.pallas import tpu as pltpu
from jax.experimental.pallas import tpu_sc as plsc

_B = 4
_N = 10000
_F_IN = 128
_DIM = 48
_TOPK = 20
_F_NON = 8
_EPS = 1e-5

_ROWS = 200


_NSEG = 40
_SEGW = _N // _NSEG
_NCH = 400
_CHW = _N // _NCH


def _cos_topk_body(emb_blk_ref, embT_ref, idx_ref):
    e = emb_blk_ref[...]
    et = embT_ref[...]
    dot = jax.lax.dot_general(
        e, et, (((1,), (0,)), ((), ())), preferred_element_type=jnp.float32)
    nrm_blk = jnp.sqrt(jnp.sum(e * e, axis=1, keepdims=True))
    nrm_all = jnp.sqrt(jnp.sum(et * et, axis=0, keepdims=True))
    cos = dot / (nrm_blk * nrm_all)

    half = _N // 2
    a = cos[:, :half]
    bb = cos[:, half:]
    c = jax.lax.broadcasted_iota(jnp.int32, (_ROWS, half), 1)
    bwin = bb > a
    M = jnp.where(bwin, bb, a)
    Mi = jnp.where(bwin, c + half, c)
    L = jnp.where(bwin, a, bb)
    Li = jnp.where(bwin, c, c + half)
    for k in range(_TOPK):
        m = jnp.max(M, axis=1, keepdims=True)
        eqm = M == m
        amin = jnp.min(jnp.where(eqm, Mi, _N), axis=1, keepdims=True)
        idx_ref[:, k] = amin[:, 0]
        sel = eqm & (Mi == amin)
        M = jnp.where(sel, L, M)
        Mi = jnp.where(sel, Li, Mi)
        L = jnp.where(sel, -jnp.inf, L)


def _cos_topk(emb):
    embT = emb.T
    return pl.pallas_call(
        _cos_topk_body,
        grid=(_N // _ROWS,),
        in_specs=[
            pl.BlockSpec((_ROWS, _DIM), lambda i: (i, 0)),
            pl.BlockSpec((_DIM, _N), lambda i: (0, 0)),
        ],
        out_specs=pl.BlockSpec((_ROWS, _TOPK), lambda i: (i, 0)),
        out_shape=jax.ShapeDtypeStruct((_N, _TOPK), jnp.int32),
    )(emb, embT)


_NW = 32
_WPB = 8
_CHUNK = 1280
_CHUNK_LAST = 1040
_NPAD = _WPB * _CHUNK
_G = 16
_K1 = _TOPK + 1


def _sc_mp_body(tk_hbm, si_hbm, sj_hbm, xl_hbm, out_hbm,
                sjb_v, sib_v, tk_v, idx_v, l_v, e_v, inv_v, g_v, o_v, sem):
    w = lax.axis_index("c") * 16 + lax.axis_index("s")
    b = w // _WPB
    c = w % _WPB
    node0 = c * _CHUNK
    cnt = jnp.where(c == _WPB - 1, _CHUNK_LAST, _CHUNK)
    boff = b * _N

    pltpu.sync_copy(sj_hbm.at[pl.ds(b * _NPAD, _NPAD)], sjb_v)
    pltpu.sync_copy(si_hbm.at[pl.ds(b * _NPAD + node0, _CHUNK)], sib_v)
    pltpu.sync_copy(tk_hbm.at[pl.ds(node0 * _TOPK, _CHUNK * _TOPK)], tk_v)

    lanes = lax.broadcasted_iota(jnp.int32, (_G,), 0)
    neg = jnp.full((_G,), -1e9, dtype=jnp.float32)

    def group(g, _):
        off = g * _G
        node16 = off + lanes
        local = node0 + node16
        si16 = sib_v[pl.ds(off, _G)]

        m = jnp.full((_G,), -jnp.inf, dtype=jnp.float32)
        for k in range(_K1):
            if k < _TOPK:
                jk = plsc.load_gather(tk_v, [node16 * _TOPK + k])
            else:
                jk = local
            sjk = plsc.load_gather(sjb_v, [jk])
            logit = si16 + sjk
            logit = jnp.maximum(logit, 0.2 * logit)
            if k < _TOPK:
                logit = jnp.where(jk == local, neg, logit)
            l_v[pl.ds(k * _G, _G)] = logit
            idx_v[k] = jk + boff
            m = jnp.maximum(m, logit)

        descs = [pltpu.async_copy(xl_hbm.at[idx_v.at[k]], g_v.at[k], sem)
                 for k in range(_K1)]

        s = jnp.zeros((_G,), dtype=jnp.float32)
        for k in range(_K1):
            ek = jnp.exp(l_v[pl.ds(k * _G, _G)] - m)
            e_v[pl.ds(k * _G, _G)] = ek
            s = s + ek
        inv_v[...] = 1.0 / s

        for d in descs:
            d.wait()

        def node_body(n, _):
            n_vec = jnp.full((_G,), n, dtype=jnp.int32)
            accs = [jnp.zeros((16,), dtype=jnp.float32) for _ in range(3)]
            for k in range(_K1):
                ek = plsc.load_gather(e_v, [n_vec + k * _G])
                for d in range(3):
                    r = g_v[k, n, pl.ds(d * 16, 16)]
                    accs[d] = accs[d] + ek * r
            iv = plsc.load_gather(inv_v, [n_vec])
            for d in range(3):
                o_v[n, pl.ds(d * 16, 16)] = accs[d] * iv
            return 0

        lax.fori_loop(0, _G, node_body, 0)
        pltpu.sync_copy(o_v, out_hbm.at[pl.ds(boff + node0 + off, _G)])
        return 0

    lax.fori_loop(0, cnt // _G, group, 0)


def _sc_message_pass(tk_pad, si_pad, sj_pad, xl_flat):
    mesh = plsc.VectorSubcoreMesh(core_axis_name="c", subcore_axis_name="s")
    f = pl.kernel(
        _sc_mp_body,
        out_type=jax.ShapeDtypeStruct((_B * _N, _DIM), jnp.float32),
        mesh=mesh,
        scratch_types=[
            pltpu.VMEM((_NPAD,), jnp.float32),
            pltpu.VMEM((_CHUNK,), jnp.float32),
            pltpu.VMEM((_CHUNK * _TOPK,), jnp.int32),
            pltpu.VMEM((_K1, _G), jnp.int32),
            pltpu.VMEM((_K1 * _G,), jnp.float32),
            pltpu.VMEM((_K1 * _G,), jnp.float32),
            pltpu.VMEM((_G,), jnp.float32),
            pltpu.VMEM((_K1, _G, 128), jnp.float32),
            pltpu.VMEM((_G, _DIM), jnp.float32),
            pltpu.SemaphoreType.DMA,
        ],
        compiler_params=pltpu.CompilerParams(needs_layout_passes=False),
    )
    return f(tk_pad, si_pad, sj_pad, xl_flat)


_RP = 400


def _post_body(agg_ref, emb_ref, xn_ref, vec_ref, w1a_ref, w1b_ref,
               w2_ref, out_ref):
    inv = 1.0 / jnp.sqrt(1.0 + _EPS)
    v = vec_ref[...]
    gnn_bias, bn1_g, bn1_b, bn2_g, bn2_b = (v[0], v[1], v[2], v[3], v[4])
    h = agg_ref[0] + gnn_bias[None, :]
    h = jax.nn.relu(bn1_g[None, :] * (h * inv) + bn1_b[None, :])
    h = h * emb_ref[...]
    h = jax.nn.relu(bn2_g[None, :] * (h * inv) + bn2_b[None, :])
    t1 = jax.lax.dot_general(h, w1a_ref[...], (((1,), (0,)), ((), ())),
                             preferred_element_type=jnp.float32)
    t2 = jax.lax.dot_general(xn_ref[...], w1b_ref[...], (((1,), (0,)), ((), ())),
                             preferred_element_type=jnp.float32)
    fc1_b = vec_ref[5, :10]
    fc2_b = vec_ref[6, :3]
    hh = jax.nn.relu(t1 + t2 + fc1_b[None, :])
    out = jax.lax.dot_general(hh, w2_ref[...], (((1,), (0,)), ((), ())),
                              preferred_element_type=jnp.float32)
    out_ref[0] = out + fc2_b[None, :]


def _postproc(agg, emb, x_non, vecs, w1a, w1b, w2):
    return pl.pallas_call(
        _post_body,
        grid=(_B, _N // _RP),
        in_specs=[
            pl.BlockSpec((1, _RP, _DIM), lambda b, j: (b, j, 0)),
            pl.BlockSpec((_RP, _DIM), lambda b, j: (j, 0)),
            pl.BlockSpec((_RP, _F_NON), lambda b, j: (j, 0)),
            pl.BlockSpec((8, _DIM), lambda b, j: (0, 0)),
            pl.BlockSpec((_DIM, 10), lambda b, j: (0, 0)),
            pl.BlockSpec((_F_NON, 10), lambda b, j: (0, 0)),
            pl.BlockSpec((10, 3), lambda b, j: (0, 0)),
        ],
        out_specs=pl.BlockSpec((1, _RP, 3), lambda b, j: (b, j, 0)),
        out_shape=jax.ShapeDtypeStruct((_B, _N, 3), jnp.float32),
    )(agg, emb, x_non, vecs, w1a, w1b, w2)


def kernel(data, org_edge_index, emb, lin_W, att_i, att_j, att_em_i,
           att_em_j, gnn_bias, bn1_g, bn1_b, bn2_g, bn2_b, fc1_W, fc1_b,
           fc2_W, fc2_b, x_non):
    topk_idx = _cos_topk(emb)
    tk_pad = jnp.pad(topk_idx, ((0, _NPAD - _N), (0, 0))).reshape(-1)

    x = data.reshape(_B * _N, _F_IN)
    xl_flat = x @ lin_W
    xl = xl_flat.reshape(_B, _N, _DIM)
    si = xl @ att_i + (emb @ att_em_i)[None, :]
    sj = xl @ att_j + (emb @ att_em_j)[None, :]
    si_pad = jnp.pad(si, ((0, 0), (0, _NPAD - _N))).reshape(-1)
    sj_pad = jnp.pad(sj, ((0, 0), (0, _NPAD - _N))).reshape(-1)

    xl_wide = jnp.pad(xl_flat, ((0, 0), (0, 128 - _DIM)))
    agg = _sc_message_pass(tk_pad, si_pad, sj_pad, xl_wide)

    out = agg.reshape(_B, _N, _DIM) + gnn_bias
    inv = 1.0 / jnp.sqrt(1.0 + _EPS)
    out = jax.nn.relu(bn1_g * (out * inv) + bn1_b)
    out = out * emb[None, :, :]
    out = jax.nn.relu(bn2_g[None, None, :] * (out * inv) + bn2_b[None, None, :])
    xn = jnp.broadcast_to(x_non[None, :, :], (_B, _N, _F_NON))
    out = jnp.concatenate([out, xn], axis=2).reshape(_B * _N, _DIM + _F_NON)
    out = jax.nn.relu(out @ fc1_W + fc1_b) @ fc2_W + fc2_b
    return out


# speedup vs baseline: 1.1052x; 1.1052x over previous
import functools

import jax
import jax.numpy as jnp
from jax import lax
from jax.experimental import pallas as pl
from jax.experimental.pallas import tpu as pltpu
from jax.experimental.pallas import tpu_sc as plsc

_B = 4
_N = 10000
_F_IN = 128
_DIM = 48
_TOPK = 20
_F_NON = 8
_EPS = 1e-5

_ROWS = 200


_NSEG = 40
_SEGW = _N // _NSEG
_NCH = 400
_CHW = _N // _NCH


def _cos_topk_body(emb_blk_ref, embT_ref, idx_ref):
    e = emb_blk_ref[...]
    et = embT_ref[...]
    dot = jax.lax.dot_general(
        e, et, (((1,), (0,)), ((), ())), preferred_element_type=jnp.float32)
    nrm_blk = jnp.sqrt(jnp.sum(e * e, axis=1, keepdims=True))
    nrm_all = jnp.sqrt(jnp.sum(et * et, axis=0, keepdims=True))
    cos = dot / (nrm_blk * nrm_all)

    col = jax.lax.broadcasted_iota(jnp.int32, cos.shape, 1)
    vals = cos
    for k in range(_TOPK):
        m = jnp.max(vals, axis=1, keepdims=True)
        amin = jnp.min(jnp.where(vals == m, col, _N), axis=1, keepdims=True)
        idx_ref[:, k] = amin[:, 0]
        vals = jnp.where(col == amin, -jnp.inf, vals)


def _cos_topk(emb):
    embT = emb.T
    return pl.pallas_call(
        _cos_topk_body,
        grid=(_N // _ROWS,),
        in_specs=[
            pl.BlockSpec((_ROWS, _DIM), lambda i: (i, 0)),
            pl.BlockSpec((_DIM, _N), lambda i: (0, 0)),
        ],
        out_specs=pl.BlockSpec((_ROWS, _TOPK), lambda i: (i, 0)),
        out_shape=jax.ShapeDtypeStruct((_N, _TOPK), jnp.int32),
    )(emb, embT)


_NW = 32
_WPB = 8
_CHUNK = 1280
_CHUNK_LAST = 1040
_NPAD = _WPB * _CHUNK
_G = 16
_K1 = _TOPK + 1


def _sc_mp_body(tk_hbm, si_hbm, sj_hbm, xl_hbm, out_hbm,
                sjb_v, sib_v, tk_v, idx_v, l_v, e_v, inv_v, g_v, o_v, sem):
    w = lax.axis_index("c") * 16 + lax.axis_index("s")
    b = w // _WPB
    c = w % _WPB
    node0 = c * _CHUNK
    cnt = jnp.where(c == _WPB - 1, _CHUNK_LAST, _CHUNK)
    boff = b * _N

    pltpu.sync_copy(sj_hbm.at[pl.ds(b * _NPAD, _NPAD)], sjb_v)
    pltpu.sync_copy(si_hbm.at[pl.ds(b * _NPAD + node0, _CHUNK)], sib_v)
    pltpu.sync_copy(tk_hbm.at[pl.ds(node0 * _TOPK, _CHUNK * _TOPK)], tk_v)

    lanes = lax.broadcasted_iota(jnp.int32, (_G,), 0)
    neg = jnp.full((_G,), -1e9, dtype=jnp.float32)

    def group(g, _):
        off = g * _G
        node16 = off + lanes
        local = node0 + node16
        si16 = sib_v[pl.ds(off, _G)]

        m = jnp.full((_G,), -jnp.inf, dtype=jnp.float32)
        for k in range(_K1):
            if k < _TOPK:
                jk = plsc.load_gather(tk_v, [node16 * _TOPK + k])
            else:
                jk = local
            sjk = plsc.load_gather(sjb_v, [jk])
            logit = si16 + sjk
            logit = jnp.maximum(logit, 0.2 * logit)
            if k < _TOPK:
                logit = jnp.where(jk == local, neg, logit)
            l_v[pl.ds(k * _G, _G)] = logit
            idx_v[k] = jk + boff
            m = jnp.maximum(m, logit)

        descs = [pltpu.async_copy(xl_hbm.at[idx_v.at[k]], g_v.at[k], sem)
                 for k in range(_K1)]

        s = jnp.zeros((_G,), dtype=jnp.float32)
        for k in range(_K1):
            ek = jnp.exp(l_v[pl.ds(k * _G, _G)] - m)
            e_v[pl.ds(k * _G, _G)] = ek
            s = s + ek
        inv_v[...] = 1.0 / s

        for d in descs:
            d.wait()

        def node_body(n, _):
            n_vec = jnp.full((_G,), n, dtype=jnp.int32)
            accs = [jnp.zeros((16,), dtype=jnp.float32) for _ in range(3)]
            for k in range(_K1):
                ek = plsc.load_gather(e_v, [n_vec + k * _G])
                for d in range(3):
                    r = g_v[k, n, pl.ds(d * 16, 16)]
                    accs[d] = accs[d] + ek * r
            iv = plsc.load_gather(inv_v, [n_vec])
            for d in range(3):
                o_v[n, pl.ds(d * 16, 16)] = accs[d] * iv
            return 0

        lax.fori_loop(0, _G, node_body, 0)
        pltpu.sync_copy(o_v, out_hbm.at[pl.ds(boff + node0 + off, _G)])
        return 0

    lax.fori_loop(0, cnt // _G, group, 0)


def _sc_message_pass(tk_pad, si_pad, sj_pad, xl_flat):
    mesh = plsc.VectorSubcoreMesh(core_axis_name="c", subcore_axis_name="s")
    f = pl.kernel(
        _sc_mp_body,
        out_type=jax.ShapeDtypeStruct((_B * _N, _DIM), jnp.float32),
        mesh=mesh,
        scratch_types=[
            pltpu.VMEM((_NPAD,), jnp.float32),
            pltpu.VMEM((_CHUNK,), jnp.float32),
            pltpu.VMEM((_CHUNK * _TOPK,), jnp.int32),
            pltpu.VMEM((_K1, _G), jnp.int32),
            pltpu.VMEM((_K1 * _G,), jnp.float32),
            pltpu.VMEM((_K1 * _G,), jnp.float32),
            pltpu.VMEM((_G,), jnp.float32),
            pltpu.VMEM((_K1, _G, 128), jnp.float32),
            pltpu.VMEM((_G, _DIM), jnp.float32),
            pltpu.SemaphoreType.DMA,
        ],
        compiler_params=pltpu.CompilerParams(needs_layout_passes=False),
    )
    return f(tk_pad, si_pad, sj_pad, xl_flat)


_RP = 400


def _post_body(agg_ref, emb_ref, xn_ref, vec_ref, w1a_ref, w1b_ref,
               w2_ref, out_ref):
    inv = 1.0 / jnp.sqrt(1.0 + _EPS)
    v = vec_ref[...]
    gnn_bias, bn1_g, bn1_b, bn2_g, bn2_b = (v[0], v[1], v[2], v[3], v[4])
    h = agg_ref[0] + gnn_bias[None, :]
    h = jax.nn.relu(bn1_g[None, :] * (h * inv) + bn1_b[None, :])
    h = h * emb_ref[...]
    h = jax.nn.relu(bn2_g[None, :] * (h * inv) + bn2_b[None, :])
    t1 = jax.lax.dot_general(h, w1a_ref[...], (((1,), (0,)), ((), ())),
                             preferred_element_type=jnp.float32)
    t2 = jax.lax.dot_general(xn_ref[...], w1b_ref[...], (((1,), (0,)), ((), ())),
                             preferred_element_type=jnp.float32)
    fc1_b = vec_ref[5, :10]
    fc2_b = vec_ref[6, :3]
    hh = jax.nn.relu(t1 + t2 + fc1_b[None, :])
    out = jax.lax.dot_general(hh, w2_ref[...], (((1,), (0,)), ((), ())),
                              preferred_element_type=jnp.float32)
    out_ref[0] = out + fc2_b[None, :]


def _postproc(agg, emb, x_non, vecs, w1a, w1b, w2):
    return pl.pallas_call(
        _post_body,
        grid=(_B, _N // _RP),
        in_specs=[
            pl.BlockSpec((1, _RP, _DIM), lambda b, j: (b, j, 0)),
            pl.BlockSpec((_RP, _DIM), lambda b, j: (j, 0)),
            pl.BlockSpec((_RP, _F_NON), lambda b, j: (j, 0)),
            pl.BlockSpec((8, _DIM), lambda b, j: (0, 0)),
            pl.BlockSpec((_DIM, 10), lambda b, j: (0, 0)),
            pl.BlockSpec((_F_NON, 10), lambda b, j: (0, 0)),
            pl.BlockSpec((10, 3), lambda b, j: (0, 0)),
        ],
        out_specs=pl.BlockSpec((1, _RP, 3), lambda b, j: (b, j, 0)),
        out_shape=jax.ShapeDtypeStruct((_B, _N, 3), jnp.float32),
    )(agg, emb, x_non, vecs, w1a, w1b, w2)


def kernel(data, org_edge_index, emb, lin_W, att_i, att_j, att_em_i,
           att_em_j, gnn_bias, bn1_g, bn1_b, bn2_g, bn2_b, fc1_W, fc1_b,
           fc2_W, fc2_b, x_non):
    topk_idx = _cos_topk(emb)
    tk_pad = jnp.pad(topk_idx, ((0, _NPAD - _N), (0, 0))).reshape(-1)

    x = data.reshape(_B * _N, _F_IN)
    xl_flat = x @ lin_W
    xl = xl_flat.reshape(_B, _N, _DIM)
    si = xl @ att_i + (emb @ att_em_i)[None, :]
    sj = xl @ att_j + (emb @ att_em_j)[None, :]
    si_pad = jnp.pad(si, ((0, 0), (0, _NPAD - _N))).reshape(-1)
    sj_pad = jnp.pad(sj, ((0, 0), (0, _NPAD - _N))).reshape(-1)

    xl_wide = jnp.pad(xl_flat, ((0, 0), (0, 128 - _DIM)))
    agg = _sc_message_pass(tk_pad, si_pad, sj_pad, xl_wide)

    out = agg.reshape(_B, _N, _DIM) + gnn_bias
    inv = 1.0 / jnp.sqrt(1.0 + _EPS)
    out = jax.nn.relu(bn1_g * (out * inv) + bn1_b)
    out = out * emb[None, :, :]
    out = jax.nn.relu(bn2_g[None, None, :] * (out * inv) + bn2_b[None, None, :])
    xn = jnp.broadcast_to(x_non[None, :, :], (_B, _N, _F_NON))
    out = jnp.concatenate([out, xn], axis=2).reshape(_B * _N, _DIM + _F_NON)
    out = jax.nn.relu(out @ fc1_W + fc1_b) @ fc2_W + fc2_b
    return out


# speedup vs baseline: 1.1517x; 1.0421x over previous
import functools

import jax
import jax.numpy as jnp
from jax import lax
from jax.experimental import pallas as pl
from jax.experimental.pallas import tpu as pltpu
from jax.experimental.pallas import tpu_sc as plsc

_B = 4
_N = 10000
_F_IN = 128
_DIM = 48
_TOPK = 20
_F_NON = 8
_EPS = 1e-5

_ROWS = 200


_NSEG = 40
_SEGW = _N // _NSEG
_NCH = 400
_CHW = _N // _NCH


def _cos_topk_body(emb_blk_ref, embT_ref, idx_ref):
    e = emb_blk_ref[...]
    et = embT_ref[...]
    dot = jax.lax.dot_general(
        e, et, (((1,), (0,)), ((), ())), preferred_element_type=jnp.float32)
    nrm_blk = jnp.sqrt(jnp.sum(e * e, axis=1, keepdims=True))
    nrm_all = jnp.sqrt(jnp.sum(et * et, axis=0, keepdims=True))
    cos = dot / (nrm_blk * nrm_all)

    col = jax.lax.broadcasted_iota(jnp.int32, cos.shape, 1)
    vals = cos
    for k in range(_TOPK):
        amin = jnp.argmax(vals, axis=1).astype(jnp.int32)[:, None]
        idx_ref[:, k] = amin[:, 0]
        vals = jnp.where(col == amin, -jnp.inf, vals)


def _cos_topk(emb):
    embT = emb.T
    return pl.pallas_call(
        _cos_topk_body,
        grid=(_N // _ROWS,),
        in_specs=[
            pl.BlockSpec((_ROWS, _DIM), lambda i: (i, 0)),
            pl.BlockSpec((_DIM, _N), lambda i: (0, 0)),
        ],
        out_specs=pl.BlockSpec((_ROWS, _TOPK), lambda i: (i, 0)),
        out_shape=jax.ShapeDtypeStruct((_N, _TOPK), jnp.int32),
    )(emb, embT)


_NW = 32
_WPB = 8
_CHUNK = 1280
_CHUNK_LAST = 1040
_NPAD = _WPB * _CHUNK
_G = 16
_K1 = _TOPK + 1


def _sc_mp_body(tk_hbm, si_hbm, sj_hbm, xl_hbm, out_hbm,
                sjb_v, sib_v, tk_v, idx_v, l_v, e_v, inv_v, g_v, o_v, sem):
    w = lax.axis_index("c") * 16 + lax.axis_index("s")
    b = w // _WPB
    c = w % _WPB
    node0 = c * _CHUNK
    cnt = jnp.where(c == _WPB - 1, _CHUNK_LAST, _CHUNK)
    boff = b * _N

    pltpu.sync_copy(sj_hbm.at[pl.ds(b * _NPAD, _NPAD)], sjb_v)
    pltpu.sync_copy(si_hbm.at[pl.ds(b * _NPAD + node0, _CHUNK)], sib_v)
    pltpu.sync_copy(tk_hbm.at[pl.ds(node0 * _TOPK, _CHUNK * _TOPK)], tk_v)

    lanes = lax.broadcasted_iota(jnp.int32, (_G,), 0)
    neg = jnp.full((_G,), -1e9, dtype=jnp.float32)

    def group(g, _):
        off = g * _G
        node16 = off + lanes
        local = node0 + node16
        si16 = sib_v[pl.ds(off, _G)]

        m = jnp.full((_G,), -jnp.inf, dtype=jnp.float32)
        for k in range(_K1):
            if k < _TOPK:
                jk = plsc.load_gather(tk_v, [node16 * _TOPK + k])
            else:
                jk = local
            sjk = plsc.load_gather(sjb_v, [jk])
            logit = si16 + sjk
            logit = jnp.maximum(logit, 0.2 * logit)
            if k < _TOPK:
                logit = jnp.where(jk == local, neg, logit)
            l_v[pl.ds(k * _G, _G)] = logit
            idx_v[k] = jk + boff
            m = jnp.maximum(m, logit)

        descs = [pltpu.async_copy(xl_hbm.at[idx_v.at[k]], g_v.at[k], sem)
                 for k in range(_K1)]

        s = jnp.zeros((_G,), dtype=jnp.float32)
        for k in range(_K1):
            ek = jnp.exp(l_v[pl.ds(k * _G, _G)] - m)
            e_v[pl.ds(k * _G, _G)] = ek
            s = s + ek
        inv_v[...] = 1.0 / s

        for d in descs:
            d.wait()

        def node_body(n, _):
            n_vec = jnp.full((_G,), n, dtype=jnp.int32)
            accs = [jnp.zeros((16,), dtype=jnp.float32) for _ in range(3)]
            for k in range(_K1):
                ek = plsc.load_gather(e_v, [n_vec + k * _G])
                for d in range(3):
                    r = g_v[k, n, pl.ds(d * 16, 16)]
                    accs[d] = accs[d] + ek * r
            iv = plsc.load_gather(inv_v, [n_vec])
            for d in range(3):
                o_v[n, pl.ds(d * 16, 16)] = accs[d] * iv
            return 0

        lax.fori_loop(0, _G, node_body, 0)
        pltpu.sync_copy(o_v, out_hbm.at[pl.ds(boff + node0 + off, _G)])
        return 0

    lax.fori_loop(0, cnt // _G, group, 0)


def _sc_message_pass(tk_pad, si_pad, sj_pad, xl_flat):
    mesh = plsc.VectorSubcoreMesh(core_axis_name="c", subcore_axis_name="s")
    f = pl.kernel(
        _sc_mp_body,
        out_type=jax.ShapeDtypeStruct((_B * _N, _DIM), jnp.float32),
        mesh=mesh,
        scratch_types=[
            pltpu.VMEM((_NPAD,), jnp.float32),
            pltpu.VMEM((_CHUNK,), jnp.float32),
            pltpu.VMEM((_CHUNK * _TOPK,), jnp.int32),
            pltpu.VMEM((_K1, _G), jnp.int32),
            pltpu.VMEM((_K1 * _G,), jnp.float32),
            pltpu.VMEM((_K1 * _G,), jnp.float32),
            pltpu.VMEM((_G,), jnp.float32),
            pltpu.VMEM((_K1, _G, 128), jnp.float32),
            pltpu.VMEM((_G, _DIM), jnp.float32),
            pltpu.SemaphoreType.DMA,
        ],
        compiler_params=pltpu.CompilerParams(needs_layout_passes=False),
    )
    return f(tk_pad, si_pad, sj_pad, xl_flat)


_RP = 400


def _post_body(agg_ref, emb_ref, xn_ref, vec_ref, w1a_ref, w1b_ref,
               w2_ref, out_ref):
    inv = 1.0 / jnp.sqrt(1.0 + _EPS)
    v = vec_ref[...]
    gnn_bias, bn1_g, bn1_b, bn2_g, bn2_b = (v[0], v[1], v[2], v[3], v[4])
    h = agg_ref[0] + gnn_bias[None, :]
    h = jax.nn.relu(bn1_g[None, :] * (h * inv) + bn1_b[None, :])
    h = h * emb_ref[...]
    h = jax.nn.relu(bn2_g[None, :] * (h * inv) + bn2_b[None, :])
    t1 = jax.lax.dot_general(h, w1a_ref[...], (((1,), (0,)), ((), ())),
                             preferred_element_type=jnp.float32)
    t2 = jax.lax.dot_general(xn_ref[...], w1b_ref[...], (((1,), (0,)), ((), ())),
                             preferred_element_type=jnp.float32)
    fc1_b = vec_ref[5, :10]
    fc2_b = vec_ref[6, :3]
    hh = jax.nn.relu(t1 + t2 + fc1_b[None, :])
    out = jax.lax.dot_general(hh, w2_ref[...], (((1,), (0,)), ((), ())),
                              preferred_element_type=jnp.float32)
    out_ref[0] = out + fc2_b[None, :]


def _postproc(agg, emb, x_non, vecs, w1a, w1b, w2):
    return pl.pallas_call(
        _post_body,
        grid=(_B, _N // _RP),
        in_specs=[
            pl.BlockSpec((1, _RP, _DIM), lambda b, j: (b, j, 0)),
            pl.BlockSpec((_RP, _DIM), lambda b, j: (j, 0)),
            pl.BlockSpec((_RP, _F_NON), lambda b, j: (j, 0)),
            pl.BlockSpec((8, _DIM), lambda b, j: (0, 0)),
            pl.BlockSpec((_DIM, 10), lambda b, j: (0, 0)),
            pl.BlockSpec((_F_NON, 10), lambda b, j: (0, 0)),
            pl.BlockSpec((10, 3), lambda b, j: (0, 0)),
        ],
        out_specs=pl.BlockSpec((1, _RP, 3), lambda b, j: (b, j, 0)),
        out_shape=jax.ShapeDtypeStruct((_B, _N, 3), jnp.float32),
    )(agg, emb, x_non, vecs, w1a, w1b, w2)


def kernel(data, org_edge_index, emb, lin_W, att_i, att_j, att_em_i,
           att_em_j, gnn_bias, bn1_g, bn1_b, bn2_g, bn2_b, fc1_W, fc1_b,
           fc2_W, fc2_b, x_non):
    topk_idx = _cos_topk(emb)
    tk_pad = jnp.pad(topk_idx, ((0, _NPAD - _N), (0, 0))).reshape(-1)

    x = data.reshape(_B * _N, _F_IN)
    xl_flat = x @ lin_W
    xl = xl_flat.reshape(_B, _N, _DIM)
    si = xl @ att_i + (emb @ att_em_i)[None, :]
    sj = xl @ att_j + (emb @ att_em_j)[None, :]
    si_pad = jnp.pad(si, ((0, 0), (0, _NPAD - _N))).reshape(-1)
    sj_pad = jnp.pad(sj, ((0, 0), (0, _NPAD - _N))).reshape(-1)

    xl_wide = jnp.pad(xl_flat, ((0, 0), (0, 128 - _DIM)))
    agg = _sc_message_pass(tk_pad, si_pad, sj_pad, xl_wide)

    out = agg.reshape(_B, _N, _DIM) + gnn_bias
    inv = 1.0 / jnp.sqrt(1.0 + _EPS)
    out = jax.nn.relu(bn1_g * (out * inv) + bn1_b)
    out = out * emb[None, :, :]
    out = jax.nn.relu(bn2_g[None, None, :] * (out * inv) + bn2_b[None, None, :])
    xn = jnp.broadcast_to(x_non[None, :, :], (_B, _N, _F_NON))
    out = jnp.concatenate([out, xn], axis=2).reshape(_B * _N, _DIM + _F_NON)
    out = jax.nn.relu(out @ fc1_W + fc1_b) @ fc2_W + fc2_b
    return out


# speedup vs baseline: 1.2696x; 1.1024x over previous
import functools

import jax
import jax.numpy as jnp
from jax import lax
from jax.experimental import pallas as pl
from jax.experimental.pallas import tpu as pltpu
from jax.experimental.pallas import tpu_sc as plsc

_B = 4
_N = 10000
_F_IN = 128
_DIM = 48
_TOPK = 20
_F_NON = 8
_EPS = 1e-5

_ROWS = 400


_NSEG = 40
_SEGW = _N // _NSEG
_NCH = 400
_CHW = _N // _NCH


def _cos_topk_body(emb_blk_ref, embT_ref, idx_ref):
    e = emb_blk_ref[...]
    et = embT_ref[...]
    dot = jax.lax.dot_general(
        e, et, (((1,), (0,)), ((), ())), preferred_element_type=jnp.float32)
    nrm_blk = jnp.sqrt(jnp.sum(e * e, axis=1, keepdims=True))
    nrm_all = jnp.sqrt(jnp.sum(et * et, axis=0, keepdims=True))
    cos = dot / (nrm_blk * nrm_all)

    col = jax.lax.broadcasted_iota(jnp.int32, cos.shape, 1)
    vals = cos
    for k in range(_TOPK):
        amin = jnp.argmax(vals, axis=1).astype(jnp.int32)[:, None]
        idx_ref[:, k] = amin[:, 0]
        vals = jnp.where(col == amin, -jnp.inf, vals)


def _cos_topk(emb):
    embT = emb.T
    return pl.pallas_call(
        _cos_topk_body,
        grid=(_N // _ROWS,),
        in_specs=[
            pl.BlockSpec((_ROWS, _DIM), lambda i: (i, 0)),
            pl.BlockSpec((_DIM, _N), lambda i: (0, 0)),
        ],
        out_specs=pl.BlockSpec((_ROWS, _TOPK), lambda i: (i, 0)),
        out_shape=jax.ShapeDtypeStruct((_N, _TOPK), jnp.int32),
    )(emb, embT)


_NW = 32
_WPB = 8
_CHUNK = 1280
_CHUNK_LAST = 1040
_NPAD = _WPB * _CHUNK
_G = 16
_K1 = _TOPK + 1


def _sc_mp_body(tk_hbm, si_hbm, sj_hbm, xl_hbm, out_hbm,
                sjb_v, sib_v, tk_v, idx_v, l_v, e_v, inv_v, g_v, o_v, sem):
    w = lax.axis_index("c") * 16 + lax.axis_index("s")
    b = w // _WPB
    c = w % _WPB
    node0 = c * _CHUNK
    cnt = jnp.where(c == _WPB - 1, _CHUNK_LAST, _CHUNK)
    boff = b * _N

    pltpu.sync_copy(sj_hbm.at[pl.ds(b * _NPAD, _NPAD)], sjb_v)
    pltpu.sync_copy(si_hbm.at[pl.ds(b * _NPAD + node0, _CHUNK)], sib_v)
    pltpu.sync_copy(tk_hbm.at[pl.ds(node0 * _TOPK, _CHUNK * _TOPK)], tk_v)

    lanes = lax.broadcasted_iota(jnp.int32, (_G,), 0)
    neg = jnp.full((_G,), -1e9, dtype=jnp.float32)

    def group(g, _):
        off = g * _G
        node16 = off + lanes
        local = node0 + node16
        si16 = sib_v[pl.ds(off, _G)]

        m = jnp.full((_G,), -jnp.inf, dtype=jnp.float32)
        for k in range(_K1):
            if k < _TOPK:
                jk = plsc.load_gather(tk_v, [node16 * _TOPK + k])
            else:
                jk = local
            sjk = plsc.load_gather(sjb_v, [jk])
            logit = si16 + sjk
            logit = jnp.maximum(logit, 0.2 * logit)
            if k < _TOPK:
                logit = jnp.where(jk == local, neg, logit)
            l_v[pl.ds(k * _G, _G)] = logit
            idx_v[k] = jk + boff
            m = jnp.maximum(m, logit)

        descs = [pltpu.async_copy(xl_hbm.at[idx_v.at[k]], g_v.at[k], sem)
                 for k in range(_K1)]

        s = jnp.zeros((_G,), dtype=jnp.float32)
        for k in range(_K1):
            ek = jnp.exp(l_v[pl.ds(k * _G, _G)] - m)
            e_v[pl.ds(k * _G, _G)] = ek
            s = s + ek
        inv_v[...] = 1.0 / s

        for d in descs:
            d.wait()

        def node_body(n, _):
            n_vec = jnp.full((_G,), n, dtype=jnp.int32)
            accs = [jnp.zeros((16,), dtype=jnp.float32) for _ in range(3)]
            for k in range(_K1):
                ek = plsc.load_gather(e_v, [n_vec + k * _G])
                for d in range(3):
                    r = g_v[k, n, pl.ds(d * 16, 16)]
                    accs[d] = accs[d] + ek * r
            iv = plsc.load_gather(inv_v, [n_vec])
            for d in range(3):
                o_v[n, pl.ds(d * 16, 16)] = accs[d] * iv
            return 0

        lax.fori_loop(0, _G, node_body, 0)
        pltpu.sync_copy(o_v, out_hbm.at[pl.ds(boff + node0 + off, _G)])
        return 0

    lax.fori_loop(0, cnt // _G, group, 0)


def _sc_message_pass(tk_pad, si_pad, sj_pad, xl_flat):
    mesh = plsc.VectorSubcoreMesh(core_axis_name="c", subcore_axis_name="s")
    f = pl.kernel(
        _sc_mp_body,
        out_type=jax.ShapeDtypeStruct((_B * _N, _DIM), jnp.float32),
        mesh=mesh,
        scratch_types=[
            pltpu.VMEM((_NPAD,), jnp.float32),
            pltpu.VMEM((_CHUNK,), jnp.float32),
            pltpu.VMEM((_CHUNK * _TOPK,), jnp.int32),
            pltpu.VMEM((_K1, _G), jnp.int32),
            pltpu.VMEM((_K1 * _G,), jnp.float32),
            pltpu.VMEM((_K1 * _G,), jnp.float32),
            pltpu.VMEM((_G,), jnp.float32),
            pltpu.VMEM((_K1, _G, 128), jnp.float32),
            pltpu.VMEM((_G, _DIM), jnp.float32),
            pltpu.SemaphoreType.DMA,
        ],
        compiler_params=pltpu.CompilerParams(needs_layout_passes=False),
    )
    return f(tk_pad, si_pad, sj_pad, xl_flat)


_RP = 400


def _post_body(agg_ref, emb_ref, xn_ref, vec_ref, w1a_ref, w1b_ref,
               w2_ref, out_ref):
    inv = 1.0 / jnp.sqrt(1.0 + _EPS)
    v = vec_ref[...]
    gnn_bias, bn1_g, bn1_b, bn2_g, bn2_b = (v[0], v[1], v[2], v[3], v[4])
    h = agg_ref[0] + gnn_bias[None, :]
    h = jax.nn.relu(bn1_g[None, :] * (h * inv) + bn1_b[None, :])
    h = h * emb_ref[...]
    h = jax.nn.relu(bn2_g[None, :] * (h * inv) + bn2_b[None, :])
    t1 = jax.lax.dot_general(h, w1a_ref[...], (((1,), (0,)), ((), ())),
                             preferred_element_type=jnp.float32)
    t2 = jax.lax.dot_general(xn_ref[...], w1b_ref[...], (((1,), (0,)), ((), ())),
                             preferred_element_type=jnp.float32)
    fc1_b = vec_ref[5, :10]
    fc2_b = vec_ref[6, :3]
    hh = jax.nn.relu(t1 + t2 + fc1_b[None, :])
    out = jax.lax.dot_general(hh, w2_ref[...], (((1,), (0,)), ((), ())),
                              preferred_element_type=jnp.float32)
    out_ref[0] = out + fc2_b[None, :]


def _postproc(agg, emb, x_non, vecs, w1a, w1b, w2):
    return pl.pallas_call(
        _post_body,
        grid=(_B, _N // _RP),
        in_specs=[
            pl.BlockSpec((1, _RP, _DIM), lambda b, j: (b, j, 0)),
            pl.BlockSpec((_RP, _DIM), lambda b, j: (j, 0)),
            pl.BlockSpec((_RP, _F_NON), lambda b, j: (j, 0)),
            pl.BlockSpec((8, _DIM), lambda b, j: (0, 0)),
            pl.BlockSpec((_DIM, 10), lambda b, j: (0, 0)),
            pl.BlockSpec((_F_NON, 10), lambda b, j: (0, 0)),
            pl.BlockSpec((10, 3), lambda b, j: (0, 0)),
        ],
        out_specs=pl.BlockSpec((1, _RP, 3), lambda b, j: (b, j, 0)),
        out_shape=jax.ShapeDtypeStruct((_B, _N, 3), jnp.float32),
    )(agg, emb, x_non, vecs, w1a, w1b, w2)


def kernel(data, org_edge_index, emb, lin_W, att_i, att_j, att_em_i,
           att_em_j, gnn_bias, bn1_g, bn1_b, bn2_g, bn2_b, fc1_W, fc1_b,
           fc2_W, fc2_b, x_non):
    topk_idx = _cos_topk(emb)
    tk_pad = jnp.pad(topk_idx, ((0, _NPAD - _N), (0, 0))).reshape(-1)

    x = data.reshape(_B * _N, _F_IN)
    xl_flat = x @ lin_W
    xl = xl_flat.reshape(_B, _N, _DIM)
    si = xl @ att_i + (emb @ att_em_i)[None, :]
    sj = xl @ att_j + (emb @ att_em_j)[None, :]
    si_pad = jnp.pad(si, ((0, 0), (0, _NPAD - _N))).reshape(-1)
    sj_pad = jnp.pad(sj, ((0, 0), (0, _NPAD - _N))).reshape(-1)

    xl_wide = jnp.pad(xl_flat, ((0, 0), (0, 128 - _DIM)))
    agg = _sc_message_pass(tk_pad, si_pad, sj_pad, xl_wide)

    out = agg.reshape(_B, _N, _DIM) + gnn_bias
    inv = 1.0 / jnp.sqrt(1.0 + _EPS)
    out = jax.nn.relu(bn1_g * (out * inv) + bn1_b)
    out = out * emb[None, :, :]
    out = jax.nn.relu(bn2_g[None, None, :] * (out * inv) + bn2_b[None, None, :])
    xn = jnp.broadcast_to(x_non[None, :, :], (_B, _N, _F_NON))
    out = jnp.concatenate([out, xn], axis=2).reshape(_B * _N, _DIM + _F_NON)
    out = jax.nn.relu(out @ fc1_W + fc1_b) @ fc2_W + fc2_b
    return out
